# Initial kernel scaffold; baseline (speedup 1.0000x reference)
import jax
import jax.numpy as jnp
from jax.experimental import pallas as pl


def kernel(x, c, Wq, Wk, Wv, Wo, W1, b1, W2, b2, g1, be1, g2, be2):
    raise NotImplementedError("write your pallas kernel here")


# speedup vs baseline: 1.8697x; 1.8697x over previous
import functools
import math

import jax
import jax.numpy as jnp
import numpy as np
from jax.experimental import pallas as pl

_B, _N, _D, _H = 1, 4096, 768, 12
_DH = _D // _H
_K = 512
_N_TOP = math.ceil(_K * 0.8)
_N_RAND = _K - _N_TOP
_DFF = 4 * _D
_ROWS = 256
_HPB = 2

_TARGET_CACHE = None


def _target_ranks() -> np.ndarray:
    global _TARGET_CACHE
    if _TARGET_CACHE is None:
        with jax.ensure_compile_time_eval():
            perm = jax.random.permutation(
                jax.random.key(1234), jnp.arange(_N - _N_TOP, dtype=jnp.int32))
            pos = np.asarray(perm[:_N_RAND])
        _TARGET_CACHE = np.concatenate(
            [np.arange(_N_TOP, dtype=np.int32), (_N_TOP + pos).astype(np.int32)])
    return _TARGET_CACHE


def _rank_body(crow_ref, call_ref, rank_ref):
    g = pl.program_id(0)
    ci = crow_ref[0, :].reshape(_ROWS, 1)
    cj = call_ref[0, :].reshape(1, _N)
    ii = jax.lax.broadcasted_iota(jnp.int32, (_ROWS, _N), 0) + g * _ROWS
    jj = jax.lax.broadcasted_iota(jnp.int32, (_ROWS, _N), 1)
    before = (cj > ci) | ((cj == ci) & (jj < ii))
    rank_ref[0, :] = jnp.sum(before.astype(jnp.int32), axis=1)


def _gather_body(rank_ref, targ_ref, x_ref, topk_ref):
    r = rank_ref[0, :].reshape(1, _N)
    t = targ_ref[0, :].reshape(_K, 1)
    onehot = (r == t).astype(jnp.float32)
    topk_ref[...] = jnp.dot(onehot, x_ref[...], preferred_element_type=jnp.float32)


def _kv_body(x_ref, g_ref, b_ref, wk_ref, wv_ref, k_ref, v_ref):
    xb = x_ref[...]
    mu = jnp.mean(xb, axis=1, keepdims=True)
    xc = xb - mu
    var = jnp.mean(xc * xc, axis=1, keepdims=True)
    xn = xc * jax.lax.rsqrt(var + 1e-5) * g_ref[0, :].reshape(1, _D) \
        + b_ref[0, :].reshape(1, _D)
    k_ref[...] = jnp.dot(xn, wk_ref[...], preferred_element_type=jnp.float32)
    v_ref[...] = jnp.dot(xn, wv_ref[...], preferred_element_type=jnp.float32)


def _attn_body(tk_ref, wq_ref, k_ref, v_ref, attn_ref, ctx_ref):
    q = jnp.dot(tk_ref[...], wq_ref[...], preferred_element_type=jnp.float32)
    for i in range(_HPB):
        sl = slice(i * _DH, (i + 1) * _DH)
        s = jax.lax.dot_general(
            q[:, sl], k_ref[:, sl], (((1,), (1,)), ((), ())),
            preferred_element_type=jnp.float32) * (1.0 / math.sqrt(_DH))
        m = jnp.max(s, axis=1, keepdims=True)
        e = jnp.exp(s - m)
        p = e / jnp.sum(e, axis=1, keepdims=True)
        attn_ref[i, ...] = p
        ctx_ref[:, sl] = jnp.dot(p, v_ref[:, sl],
                                 preferred_element_type=jnp.float32)


def _ff_body(tk_ref, ctx_ref, wo_ref, w1_ref, b1_ref, w2_ref, b2_ref,
             g2_ref, be2_ref, out_ref):
    x1 = tk_ref[...] + jnp.dot(ctx_ref[...], wo_ref[...],
                               preferred_element_type=jnp.float32)
    mu = jnp.mean(x1, axis=1, keepdims=True)
    xc = x1 - mu
    var = jnp.mean(xc * xc, axis=1, keepdims=True)
    xn = xc * jax.lax.rsqrt(var + 1e-5) * g2_ref[0, :].reshape(1, _D) \
        + be2_ref[0, :].reshape(1, _D)
    h = jnp.maximum(
        jnp.dot(xn, w1_ref[...], preferred_element_type=jnp.float32)
        + b1_ref[0, :].reshape(1, _DFF), 0.0)
    out_ref[...] = x1 + jnp.dot(h, w2_ref[...], preferred_element_type=jnp.float32) \
        + b2_ref[0, :].reshape(1, _D)


def kernel(x, c, Wq, Wk, Wv, Wo, W1, b1, W2, b2, g1, be1, g2, be2):
    x2d = x[0]
    c2d = c[0, :, 0].reshape(1, _N)

    ranks = pl.pallas_call(
        _rank_body,
        grid=(_N // _ROWS,),
        in_specs=[
            pl.BlockSpec((1, _ROWS), lambda g: (0, g)),
            pl.BlockSpec((1, _N), lambda g: (0, 0)),
        ],
        out_specs=pl.BlockSpec((1, _ROWS), lambda g: (0, g)),
        out_shape=jax.ShapeDtypeStruct((1, _N), jnp.int32),
    )(c2d, c2d)

    targ = jnp.asarray(_target_ranks()).reshape(1, _K)
    topk = pl.pallas_call(
        _gather_body,
        in_specs=[
            pl.BlockSpec((1, _N), lambda: (0, 0)),
            pl.BlockSpec((1, _K), lambda: (0, 0)),
            pl.BlockSpec((_N, _D), lambda: (0, 0)),
        ],
        out_specs=pl.BlockSpec((_K, _D), lambda: (0, 0)),
        out_shape=jax.ShapeDtypeStruct((_K, _D), jnp.float32),
    )(ranks, targ, x2d)

    kv_rows = 512
    k, v = pl.pallas_call(
        _kv_body,
        grid=(_N // kv_rows,),
        in_specs=[
            pl.BlockSpec((kv_rows, _D), lambda g: (g, 0)),
            pl.BlockSpec((1, _D), lambda g: (0, 0)),
            pl.BlockSpec((1, _D), lambda g: (0, 0)),
            pl.BlockSpec((_D, _D), lambda g: (0, 0)),
            pl.BlockSpec((_D, _D), lambda g: (0, 0)),
        ],
        out_specs=[
            pl.BlockSpec((kv_rows, _D), lambda g: (g, 0)),
            pl.BlockSpec((kv_rows, _D), lambda g: (g, 0)),
        ],
        out_shape=[
            jax.ShapeDtypeStruct((_N, _D), jnp.float32),
            jax.ShapeDtypeStruct((_N, _D), jnp.float32),
        ],
    )(x2d, g1.reshape(1, _D), be1.reshape(1, _D), Wk, Wv)

    attn, ctx = pl.pallas_call(
        _attn_body,
        grid=(_H // _HPB,),
        in_specs=[
            pl.BlockSpec((_K, _D), lambda h: (0, 0)),
            pl.BlockSpec((_D, _HPB * _DH), lambda h: (0, h)),
            pl.BlockSpec((_N, _HPB * _DH), lambda h: (0, h)),
            pl.BlockSpec((_N, _HPB * _DH), lambda h: (0, h)),
        ],
        out_specs=[
            pl.BlockSpec((_HPB, _K, _N), lambda h: (h, 0, 0)),
            pl.BlockSpec((_K, _HPB * _DH), lambda h: (0, h)),
        ],
        out_shape=[
            jax.ShapeDtypeStruct((_H, _K, _N), jnp.float32),
            jax.ShapeDtypeStruct((_K, _D), jnp.float32),
        ],
    )(topk, Wq, k, v)

    x2 = pl.pallas_call(
        _ff_body,
        in_specs=[
            pl.BlockSpec((_K, _D), lambda: (0, 0)),
            pl.BlockSpec((_K, _D), lambda: (0, 0)),
            pl.BlockSpec((_D, _D), lambda: (0, 0)),
            pl.BlockSpec((_D, _DFF), lambda: (0, 0)),
            pl.BlockSpec((1, _DFF), lambda: (0, 0)),
            pl.BlockSpec((_DFF, _D), lambda: (0, 0)),
            pl.BlockSpec((1, _D), lambda: (0, 0)),
            pl.BlockSpec((1, _D), lambda: (0, 0)),
            pl.BlockSpec((1, _D), lambda: (0, 0)),
        ],
        out_specs=pl.BlockSpec((_K, _D), lambda: (0, 0)),
        out_shape=jax.ShapeDtypeStruct((_K, _D), jnp.float32),
    )(topk, ctx, Wo, W1, b1.reshape(1, _DFF), W2, b2.reshape(1, _D),
      g2.reshape(1, _D), be2.reshape(1, _D))

    return x2[None], attn[None]
